# baseline (speedup 1.0000x reference)
import functools

import jax
import jax.numpy as jnp
from jax import lax
from jax.experimental import pallas as pl
from jax.experimental.pallas import tpu as pltpu
from jax.experimental.pallas import tpu_sc as plsc

N = 50000
C = 128
K = 27
DD = 66
T = DD * DD * DD * 2
T_PAD = 576_000
TRASH = T

NW = 32
SUB = 128
N_PAD_A = 53248
CHUNK_Q = N_PAD_A // NW
NSUB_Q = CHUNK_Q // SUB
CHUNK_S = N_PAD_A // 16
NSUB_S = CHUNK_S // SUB
TBL_WORDS = 2 * T_PAD
MEMSET_W = TBL_WORDS // NW
MBUF = 4000

BN_B = 512
N_PAD_B = 50688
Z_ROWS = K * N_PAD_B
INVALID_ROW = K * N

_DELTAS = [dx * DD * DD + dy * DD + dz
           for dx in (-1, 0, 1) for dy in (-1, 0, 1) for dz in (-1, 0, 1)]

_mesh = plsc.VectorSubcoreMesh(core_axis_name="c", subcore_axis_name="s")


def _key16(cb, cx, cy, cz, sl):
    b = cb[sl]
    x = cx[sl]
    y = cy[sl]
    z = cz[sl]
    return ((b * DD + x + 1) * DD + y + 1) * DD + z + 1


@functools.partial(
    pl.kernel,
    out_type=(
        jax.ShapeDtypeStruct((N_PAD_A // SUB, K, SUB), jnp.int32),
        jax.ShapeDtypeStruct((TBL_WORDS,), jnp.int32),
    ),
    mesh=_mesh,
    scratch_types=[
        pltpu.VMEM((CHUNK_S,), jnp.int32),
        pltpu.VMEM((CHUNK_S,), jnp.int32),
        pltpu.VMEM((CHUNK_S,), jnp.int32),
        pltpu.VMEM((CHUNK_S,), jnp.int32),
        pltpu.VMEM((NSUB_S, SUB), jnp.int32),
        pltpu.VMEM((NSUB_S, SUB), jnp.int32),
        pltpu.VMEM((K, SUB), jnp.int32),
        pltpu.VMEM((K, SUB), jnp.int32),
        pltpu.VMEM((MBUF,), jnp.int32),
        pltpu.SemaphoreType.DMA,
    ],
)
def _neighbors_kernel(cb, cx, cy, cz, srow_out, table_out,
                      cbv, cxv, cyv, czv, kidx, vals, qbuf, tbuf, mbuf, sem):
    core = lax.axis_index("c")
    sub = lax.axis_index("s")
    wid = core * 16 + sub
    core_off = core * T_PAD
    iota = lax.iota(jnp.int32, 16)

    neg1 = jnp.full((16,), -1, jnp.int32)

    def _fill(i, _):
        mbuf[pl.ds(i * 16, 16)] = neg1
        return ()
    lax.fori_loop(0, MBUF // 16, _fill, ())
    mbase = wid * MEMSET_W

    def _memset(m, _):
        pltpu.sync_copy(mbuf, table_out.at[pl.ds(mbase + m * MBUF, MBUF)])
        return ()
    lax.fori_loop(0, MEMSET_W // MBUF, _memset, ())

    sbase = sub * CHUNK_S
    pltpu.sync_copy(cb.at[pl.ds(sbase, CHUNK_S)], cbv)
    pltpu.sync_copy(cx.at[pl.ds(sbase, CHUNK_S)], cxv)
    pltpu.sync_copy(cy.at[pl.ds(sbase, CHUNK_S)], cyv)
    pltpu.sync_copy(cz.at[pl.ds(sbase, CHUNK_S)], czv)

    def _build(j, _):
        for t in range(SUB // 16):
            sl = pl.ds(j * SUB + t * 16, 16)
            key = _key16(cbv, cxv, cyv, czv, sl) + core_off
            gi = sbase + j * SUB + t * 16 + iota
            tsl = pl.ds(t * 16, 16)
            kidx[j, tsl] = jnp.where(gi < N, key, core_off + TRASH)
            vals[j, tsl] = gi
        pltpu.sync_copy(vals.at[j], table_out.at[kidx.at[j]])
        return ()
    lax.fori_loop(0, NSUB_S, _build, ())

    plsc.subcore_barrier()

    wbase = wid * CHUNK_Q
    pltpu.sync_copy(cb.at[pl.ds(wbase, CHUNK_Q)], cbv.at[pl.ds(0, CHUNK_Q)])
    pltpu.sync_copy(cx.at[pl.ds(wbase, CHUNK_Q)], cxv.at[pl.ds(0, CHUNK_Q)])
    pltpu.sync_copy(cy.at[pl.ds(wbase, CHUNK_Q)], cyv.at[pl.ds(0, CHUNK_Q)])
    pltpu.sync_copy(cz.at[pl.ds(wbase, CHUNK_Q)], czv.at[pl.ds(0, CHUNK_Q)])

    def _query(j, _):
        for t in range(SUB // 16):
            sl = pl.ds(j * SUB + t * 16, 16)
            key = _key16(cbv, cxv, cyv, czv, sl) + core_off
            tsl = pl.ds(t * 16, 16)
            for k in range(K):
                qbuf[k, tsl] = key + _DELTAS[k]
        handles = [
            pltpu.async_copy(table_out.at[qbuf.at[k]], tbuf.at[k], sem)
            for k in range(K)
        ]
        for h in handles:
            h.wait()
        for t in range(SUB // 16):
            tsl = pl.ds(t * 16, 16)
            gi = wbase + j * SUB + t * 16 + iota
            pad = gi >= N
            for k in range(K):
                v = tbuf[k, tsl]
                tbuf[k, tsl] = jnp.where(
                    pad | (v < 0), INVALID_ROW, v * K + k)
        pltpu.sync_copy(tbuf, srow_out.at[wid * NSUB_Q + j])
        return ()
    lax.fori_loop(0, NSUB_Q, _query, ())


def _gemm_body(x_ref, w_ref, o_ref):
    o_ref[...] = jnp.dot(x_ref[...], w_ref[...],
                         preferred_element_type=jnp.float32)


def _gemm(feats_pad, wcat):
    return pl.pallas_call(
        _gemm_body,
        grid=(N_PAD_B // BN_B,),
        in_specs=[
            pl.BlockSpec((BN_B, C), lambda i: (i, 0)),
            pl.BlockSpec((C, K * C), lambda i: (0, 0)),
        ],
        out_specs=pl.BlockSpec((BN_B, K * C), lambda i: (i, 0)),
        out_shape=jax.ShapeDtypeStruct((N_PAD_B, K * C), jnp.float32),
    )(feats_pad, wcat)


PG = 32


@functools.partial(
    pl.kernel,
    out_type=jax.ShapeDtypeStruct((N_PAD_A, C), jnp.float32),
    mesh=_mesh,
    scratch_types=[
        pltpu.VMEM((K, SUB), jnp.int32),
        pltpu.VMEM((K, PG, C), jnp.float32),
        pltpu.VMEM((PG, C), jnp.float32),
        pltpu.SemaphoreType.DMA,
    ],
)
def _gather_sum_kernel(srow, z, out, sidx, gbuf, acc, sem):
    core = lax.axis_index("c")
    sub = lax.axis_index("s")
    wid = core * 16 + sub

    def _block(j, _):
        jg = wid * NSUB_Q + j
        pltpu.sync_copy(srow.at[jg], sidx)
        for g in range(SUB // PG):
            handles = [
                pltpu.async_copy(z.at[sidx.at[k, pl.ds(g * PG, PG)]],
                                 gbuf.at[k], sem)
                for k in range(K)
            ]
            for h in handles:
                h.wait()

            def _row(r, _):
                for t in range(C // 16):
                    tsl = pl.ds(t * 16, 16)
                    v = gbuf[0, r, tsl]
                    for k in range(1, K):
                        v = v + gbuf[k, r, tsl]
                    acc[r, tsl] = v
                return ()
            lax.fori_loop(0, PG, _row, ())
            pltpu.sync_copy(
                acc, out.at[pl.ds(jg * SUB + g * PG, PG)])
        return ()
    lax.fori_loop(0, NSUB_Q, _block, ())


BN_D = 2048


def _stats_body(x_ref, o_ref):
    i = pl.program_id(0)

    @pl.when(i == 0)
    def _():
        o_ref[...] = jnp.zeros_like(o_ref)
    x = x_ref[...]
    o_ref[0:1, :] += jnp.sum(x, axis=0, keepdims=True)
    o_ref[1:2, :] += jnp.sum(x * x, axis=0, keepdims=True)


def _stats(out_pre):
    return pl.pallas_call(
        _stats_body,
        grid=(N_PAD_A // BN_D,),
        in_specs=[pl.BlockSpec((BN_D, C), lambda i: (i, 0))],
        out_specs=pl.BlockSpec((8, C), lambda i: (0, 0)),
        out_shape=jax.ShapeDtypeStruct((8, C), jnp.float32),
    )(out_pre)


def _bn_body(x_ref, s_ref, g_ref, b_ref, o_ref):
    x = x_ref[...]
    mean = s_ref[0:1, :] * (1.0 / N)
    var = s_ref[1:2, :] * (1.0 / N) - mean * mean
    rstd = lax.rsqrt(var + 1e-5)
    y = (x - mean) * (rstd * g_ref[...]) + b_ref[...]
    o_ref[...] = jnp.maximum(y, 0.0)


def _bn_relu(out_pre, stats, gamma, beta):
    return pl.pallas_call(
        _bn_body,
        grid=(N_PAD_A // BN_D,),
        in_specs=[
            pl.BlockSpec((BN_D, C), lambda i: (i, 0)),
            pl.BlockSpec((8, C), lambda i: (0, 0)),
            pl.BlockSpec((1, C), lambda i: (0, 0)),
            pl.BlockSpec((1, C), lambda i: (0, 0)),
        ],
        out_specs=pl.BlockSpec((BN_D, C), lambda i: (i, 0)),
        out_shape=jax.ShapeDtypeStruct((N_PAD_A, C), jnp.float32),
    )(out_pre, stats, gamma, beta)


def kernel(feats, coords, W, gamma, beta):
    ci = coords.astype(jnp.int32)
    cpad = jnp.pad(ci, ((0, N_PAD_A - N), (0, 0)))
    ct = cpad.T
    cb, cx, cy, cz = ct[0], ct[1], ct[2], ct[3]

    srow, _table = _neighbors_kernel(cb, cx, cy, cz)

    feats_pad = jnp.pad(feats, ((0, N_PAD_B - N), (0, 0)))
    wcat = jnp.transpose(W, (1, 0, 2)).reshape(C, K * C)
    zmat = _gemm(feats_pad, wcat)
    z = zmat.reshape(Z_ROWS, C)

    out_pre = _gather_sum_kernel(srow, z)

    stats = _stats(out_pre)
    y = _bn_relu(out_pre, stats, gamma.reshape(1, C), beta.reshape(1, C))
    return y[:N]


# speedup vs baseline: 28.5860x; 28.5860x over previous
import functools

import jax
import jax.numpy as jnp
from jax import lax
from jax.experimental import pallas as pl
from jax.experimental.pallas import tpu as pltpu
from jax.experimental.pallas import tpu_sc as plsc

N = 50000
C = 128
K = 27
DD = 66
T = DD * DD * DD * 2
T_PAD = 576_000
TRASH = T

NW = 32
SUB = 128
N_PAD_A = 53248
CHUNK_Q = N_PAD_A // NW
NSUB_Q = CHUNK_Q // SUB
CHUNK_S = N_PAD_A // 16
NSUB_S = CHUNK_S // SUB
TBL_WORDS = 2 * T_PAD
MEMSET_W = TBL_WORDS // NW
MBUF = 4000

BN_B = 512
N_PAD_B = 50688
Z_ROWS = K * N_PAD_B
INVALID_ROW = K * N

_DELTAS = [dx * DD * DD + dy * DD + dz
           for dx in (-1, 0, 1) for dy in (-1, 0, 1) for dz in (-1, 0, 1)]

_mesh = plsc.VectorSubcoreMesh(core_axis_name="c", subcore_axis_name="s")


def _key16(cb, cx, cy, cz, sl):
    b = cb[sl]
    x = cx[sl]
    y = cy[sl]
    z = cz[sl]
    return ((b * DD + x + 1) * DD + y + 1) * DD + z + 1


@functools.partial(
    pl.kernel,
    out_type=(
        jax.ShapeDtypeStruct((N_PAD_A // SUB, K, SUB), jnp.int32),
        jax.ShapeDtypeStruct((TBL_WORDS,), jnp.int32),
    ),
    mesh=_mesh,
    scratch_types=[
        pltpu.VMEM((CHUNK_S,), jnp.int32),
        pltpu.VMEM((CHUNK_S,), jnp.int32),
        pltpu.VMEM((CHUNK_S,), jnp.int32),
        pltpu.VMEM((CHUNK_S,), jnp.int32),
        pltpu.VMEM((NSUB_S, SUB), jnp.int32),
        pltpu.VMEM((NSUB_S, SUB), jnp.int32),
        pltpu.VMEM((K, SUB), jnp.int32),
        pltpu.VMEM((K, SUB), jnp.int32),
        pltpu.VMEM((MBUF,), jnp.int32),
        pltpu.SemaphoreType.DMA,
    ],
)
def _neighbors_kernel(cb, cx, cy, cz, srow_out, table_out,
                      cbv, cxv, cyv, czv, kidx, vals, qbuf, tbuf, mbuf, sem):
    core = lax.axis_index("c")
    sub = lax.axis_index("s")
    wid = core * 16 + sub
    core_off = core * T_PAD
    iota = lax.iota(jnp.int32, 16)

    neg1 = jnp.full((16,), -1, jnp.int32)

    def _fill(i, _):
        mbuf[pl.ds(i * 16, 16)] = neg1
        return ()
    lax.fori_loop(0, MBUF // 16, _fill, ())
    mbase = wid * MEMSET_W

    def _memset(m, _):
        pltpu.sync_copy(mbuf, table_out.at[pl.ds(mbase + m * MBUF, MBUF)])
        return ()
    lax.fori_loop(0, MEMSET_W // MBUF, _memset, ())

    sbase = sub * CHUNK_S
    pltpu.sync_copy(cb.at[pl.ds(sbase, CHUNK_S)], cbv)
    pltpu.sync_copy(cx.at[pl.ds(sbase, CHUNK_S)], cxv)
    pltpu.sync_copy(cy.at[pl.ds(sbase, CHUNK_S)], cyv)
    pltpu.sync_copy(cz.at[pl.ds(sbase, CHUNK_S)], czv)

    def _build(j, _):
        for t in range(SUB // 16):
            sl = pl.ds(j * SUB + t * 16, 16)
            key = _key16(cbv, cxv, cyv, czv, sl) + core_off
            gi = sbase + j * SUB + t * 16 + iota
            tsl = pl.ds(t * 16, 16)
            kidx[j, tsl] = jnp.where(gi < N, key, core_off + TRASH)
            vals[j, tsl] = gi
        pltpu.sync_copy(vals.at[j], table_out.at[kidx.at[j]])
        return ()
    lax.fori_loop(0, NSUB_S, _build, ())

    plsc.subcore_barrier()

    wbase = wid * CHUNK_Q
    pltpu.sync_copy(cb.at[pl.ds(wbase, CHUNK_Q)], cbv.at[pl.ds(0, CHUNK_Q)])
    pltpu.sync_copy(cx.at[pl.ds(wbase, CHUNK_Q)], cxv.at[pl.ds(0, CHUNK_Q)])
    pltpu.sync_copy(cy.at[pl.ds(wbase, CHUNK_Q)], cyv.at[pl.ds(0, CHUNK_Q)])
    pltpu.sync_copy(cz.at[pl.ds(wbase, CHUNK_Q)], czv.at[pl.ds(0, CHUNK_Q)])

    def _query(j, _):
        for t in range(SUB // 16):
            sl = pl.ds(j * SUB + t * 16, 16)
            key = _key16(cbv, cxv, cyv, czv, sl) + core_off
            tsl = pl.ds(t * 16, 16)
            for k in range(K):
                qbuf[k, tsl] = key + _DELTAS[k]
        handles = [
            pltpu.async_copy(table_out.at[qbuf.at[k]], tbuf.at[k], sem)
            for k in range(K)
        ]
        for h in handles:
            h.wait()
        for t in range(SUB // 16):
            tsl = pl.ds(t * 16, 16)
            gi = wbase + j * SUB + t * 16 + iota
            pad = gi >= N
            for k in range(K):
                v = tbuf[k, tsl]
                zrow = INVALID_ROW + ((gi + k * 5003) & 16383)
                tbuf[k, tsl] = jnp.where(pad | (v < 0), zrow, v * K + k)
        pltpu.sync_copy(tbuf, srow_out.at[wid * NSUB_Q + j])
        return ()
    lax.fori_loop(0, NSUB_Q, _query, ())


def _gemm_body(x_ref, w_ref, o_ref):
    o_ref[...] = jnp.dot(x_ref[...], w_ref[...],
                         preferred_element_type=jnp.float32)


def _gemm(feats_pad, wcat):
    return pl.pallas_call(
        _gemm_body,
        grid=(N_PAD_B // BN_B,),
        in_specs=[
            pl.BlockSpec((BN_B, C), lambda i: (i, 0)),
            pl.BlockSpec((C, K * C), lambda i: (0, 0)),
        ],
        out_specs=pl.BlockSpec((BN_B, K * C), lambda i: (i, 0)),
        out_shape=jax.ShapeDtypeStruct((N_PAD_B, K * C), jnp.float32),
    )(feats_pad, wcat)


PG = 32


@functools.partial(
    pl.kernel,
    out_type=jax.ShapeDtypeStruct((N_PAD_A, C), jnp.float32),
    mesh=_mesh,
    scratch_types=[
        pltpu.VMEM((K, SUB), jnp.int32),
        pltpu.VMEM((K, PG, C), jnp.float32),
        pltpu.VMEM((PG, C), jnp.float32),
        pltpu.SemaphoreType.DMA,
    ],
)
def _gather_sum_kernel(srow, z, out, sidx, gbuf, acc, sem):
    core = lax.axis_index("c")
    sub = lax.axis_index("s")
    wid = core * 16 + sub

    def _block(j, _):
        jg = wid * NSUB_Q + j
        pltpu.sync_copy(srow.at[jg], sidx)
        for g in range(SUB // PG):
            handles = [
                pltpu.async_copy(z.at[sidx.at[k, pl.ds(g * PG, PG)]],
                                 gbuf.at[k], sem)
                for k in range(K)
            ]
            for h in handles:
                h.wait()

            def _row(r, _):
                for t in range(C // 16):
                    tsl = pl.ds(t * 16, 16)
                    v = gbuf[0, r, tsl]
                    for k in range(1, K):
                        v = v + gbuf[k, r, tsl]
                    acc[r, tsl] = v
                return ()
            lax.fori_loop(0, PG, _row, ())
            pltpu.sync_copy(
                acc, out.at[pl.ds(jg * SUB + g * PG, PG)])
        return ()
    lax.fori_loop(0, NSUB_Q, _block, ())


BN_D = 2048


def _stats_body(x_ref, o_ref):
    i = pl.program_id(0)

    @pl.when(i == 0)
    def _():
        o_ref[...] = jnp.zeros_like(o_ref)
    x = x_ref[...]
    o_ref[0:1, :] += jnp.sum(x, axis=0, keepdims=True)
    o_ref[1:2, :] += jnp.sum(x * x, axis=0, keepdims=True)


def _stats(out_pre):
    return pl.pallas_call(
        _stats_body,
        grid=(N_PAD_A // BN_D,),
        in_specs=[pl.BlockSpec((BN_D, C), lambda i: (i, 0))],
        out_specs=pl.BlockSpec((8, C), lambda i: (0, 0)),
        out_shape=jax.ShapeDtypeStruct((8, C), jnp.float32),
    )(out_pre)


def _bn_body(x_ref, s_ref, g_ref, b_ref, o_ref):
    x = x_ref[...]
    mean = s_ref[0:1, :] * (1.0 / N)
    var = s_ref[1:2, :] * (1.0 / N) - mean * mean
    rstd = lax.rsqrt(var + 1e-5)
    y = (x - mean) * (rstd * g_ref[...]) + b_ref[...]
    o_ref[...] = jnp.maximum(y, 0.0)


def _bn_relu(out_pre, stats, gamma, beta):
    return pl.pallas_call(
        _bn_body,
        grid=(N_PAD_A // BN_D,),
        in_specs=[
            pl.BlockSpec((BN_D, C), lambda i: (i, 0)),
            pl.BlockSpec((8, C), lambda i: (0, 0)),
            pl.BlockSpec((1, C), lambda i: (0, 0)),
            pl.BlockSpec((1, C), lambda i: (0, 0)),
        ],
        out_specs=pl.BlockSpec((BN_D, C), lambda i: (i, 0)),
        out_shape=jax.ShapeDtypeStruct((N_PAD_A, C), jnp.float32),
    )(out_pre, stats, gamma, beta)


def kernel(feats, coords, W, gamma, beta):
    ci = coords.astype(jnp.int32)
    cpad = jnp.pad(ci, ((0, N_PAD_A - N), (0, 0)))
    ct = cpad.T
    cb, cx, cy, cz = ct[0], ct[1], ct[2], ct[3]

    srow, _table = _neighbors_kernel(cb, cx, cy, cz)

    feats_pad = jnp.pad(feats, ((0, N_PAD_B - N), (0, 0)))
    wcat = jnp.transpose(W, (1, 0, 2)).reshape(C, K * C)
    zmat = _gemm(feats_pad, wcat)
    z = zmat.reshape(Z_ROWS, C)

    out_pre = _gather_sum_kernel(srow, z)

    stats = _stats(out_pre)
    y = _bn_relu(out_pre, stats, gamma.reshape(1, C), beta.reshape(1, C))
    return y[:N]


# speedup vs baseline: 33.5176x; 1.1725x over previous
import functools

import jax
import jax.numpy as jnp
from jax import lax
from jax.experimental import pallas as pl
from jax.experimental.pallas import tpu as pltpu
from jax.experimental.pallas import tpu_sc as plsc

N = 50000
C = 128
K = 27
DD = 66
T = DD * DD * DD * 2
T_PAD = 576_000
TRASH = T

NW = 32
SUB = 128
N_PAD_A = 53248
CHUNK_Q = N_PAD_A // NW
NSUB_Q = CHUNK_Q // SUB
CHUNK_S = N_PAD_A // 16
NSUB_S = CHUNK_S // SUB
MEMSET_SP = T_PAD // 16
MBUF = 4000

BN_B = 512
N_PAD_B = 50688
Z_ROWS = K * N_PAD_B
INVALID_ROW = K * N

_DELTAS = [dx * DD * DD + dy * DD + dz
           for dx in (-1, 0, 1) for dy in (-1, 0, 1) for dz in (-1, 0, 1)]

_mesh = plsc.VectorSubcoreMesh(core_axis_name="c", subcore_axis_name="s")


def _key16(cb, cx, cy, cz, sl):
    b = cb[sl]
    x = cx[sl]
    y = cy[sl]
    z = cz[sl]
    return ((b * DD + x + 1) * DD + y + 1) * DD + z + 1


@functools.partial(
    pl.kernel,
    out_type=jax.ShapeDtypeStruct((N_PAD_A // SUB, K, SUB), jnp.int32),
    mesh=_mesh,
    scratch_types=[
        pltpu.VMEM((CHUNK_S,), jnp.int32),
        pltpu.VMEM((CHUNK_S,), jnp.int32),
        pltpu.VMEM((CHUNK_S,), jnp.int32),
        pltpu.VMEM((CHUNK_S,), jnp.int32),
        pltpu.VMEM((NSUB_S, SUB), jnp.int32),
        pltpu.VMEM((NSUB_S, SUB), jnp.int32),
        pltpu.VMEM((K, SUB), jnp.int32),
        pltpu.VMEM((K, SUB), jnp.int32),
        pltpu.VMEM((MBUF,), jnp.int32),
        pltpu.VMEM_SHARED((T_PAD,), jnp.int32),
        pltpu.SemaphoreType.DMA,
    ],
)
def _neighbors_kernel(cb, cx, cy, cz, srow_out,
                      cbv, cxv, cyv, czv, kidx, vals, qbuf, tbuf, mbuf,
                      table, sem):
    core = lax.axis_index("c")
    sub = lax.axis_index("s")
    wid = core * 16 + sub
    iota = lax.iota(jnp.int32, 16)

    neg1 = jnp.full((16,), -1, jnp.int32)

    def _fill(i, _):
        mbuf[pl.ds(i * 16, 16)] = neg1
        return ()
    lax.fori_loop(0, MBUF // 16, _fill, ())
    mbase = sub * MEMSET_SP

    def _memset(m, _):
        pltpu.sync_copy(mbuf, table.at[pl.ds(mbase + m * MBUF, MBUF)])
        return ()
    lax.fori_loop(0, MEMSET_SP // MBUF, _memset, ())

    sbase = sub * CHUNK_S
    pltpu.sync_copy(cb.at[pl.ds(sbase, CHUNK_S)], cbv)
    pltpu.sync_copy(cx.at[pl.ds(sbase, CHUNK_S)], cxv)
    pltpu.sync_copy(cy.at[pl.ds(sbase, CHUNK_S)], cyv)
    pltpu.sync_copy(cz.at[pl.ds(sbase, CHUNK_S)], czv)

    def _build(j, _):
        for t in range(SUB // 16):
            sl = pl.ds(j * SUB + t * 16, 16)
            key = _key16(cbv, cxv, cyv, czv, sl)
            gi = sbase + j * SUB + t * 16 + iota
            tsl = pl.ds(t * 16, 16)
            kidx[j, tsl] = jnp.where(gi < N, key, TRASH + (gi & 511))
            vals[j, tsl] = gi
        pltpu.sync_copy(vals.at[j], table.at[kidx.at[j]])
        return ()
    lax.fori_loop(0, NSUB_S, _build, ())

    plsc.subcore_barrier()

    wbase = wid * CHUNK_Q
    pltpu.sync_copy(cb.at[pl.ds(wbase, CHUNK_Q)], cbv.at[pl.ds(0, CHUNK_Q)])
    pltpu.sync_copy(cx.at[pl.ds(wbase, CHUNK_Q)], cxv.at[pl.ds(0, CHUNK_Q)])
    pltpu.sync_copy(cy.at[pl.ds(wbase, CHUNK_Q)], cyv.at[pl.ds(0, CHUNK_Q)])
    pltpu.sync_copy(cz.at[pl.ds(wbase, CHUNK_Q)], czv.at[pl.ds(0, CHUNK_Q)])

    def _query(j, _):
        for t in range(SUB // 16):
            sl = pl.ds(j * SUB + t * 16, 16)
            key = _key16(cbv, cxv, cyv, czv, sl)
            tsl = pl.ds(t * 16, 16)
            for k in range(K):
                qbuf[k, tsl] = key + _DELTAS[k]
        handles = [
            pltpu.async_copy(table.at[qbuf.at[k]], tbuf.at[k], sem)
            for k in range(K)
        ]
        for h in handles:
            h.wait()
        for t in range(SUB // 16):
            tsl = pl.ds(t * 16, 16)
            gi = wbase + j * SUB + t * 16 + iota
            pad = gi >= N
            for k in range(K):
                v = tbuf[k, tsl]
                zrow = INVALID_ROW + ((gi + k * 5003) & 16383)
                tbuf[k, tsl] = jnp.where(pad | (v < 0), zrow, v * K + k)
        pltpu.sync_copy(tbuf, srow_out.at[wid * NSUB_Q + j])
        return ()
    lax.fori_loop(0, NSUB_Q, _query, ())


def _gemm_body(x_ref, w_ref, o_ref):
    o_ref[...] = jnp.dot(x_ref[...], w_ref[...],
                         preferred_element_type=jnp.float32)


def _gemm(feats_pad, wcat):
    return pl.pallas_call(
        _gemm_body,
        grid=(N_PAD_B // BN_B,),
        in_specs=[
            pl.BlockSpec((BN_B, C), lambda i: (i, 0)),
            pl.BlockSpec((C, K * C), lambda i: (0, 0)),
        ],
        out_specs=pl.BlockSpec((BN_B, K * C), lambda i: (i, 0)),
        out_shape=jax.ShapeDtypeStruct((N_PAD_B, K * C), jnp.float32),
    )(feats_pad, wcat)


PG = 32


@functools.partial(
    pl.kernel,
    out_type=jax.ShapeDtypeStruct((N_PAD_A, C), jnp.float32),
    mesh=_mesh,
    scratch_types=[
        pltpu.VMEM((K, SUB), jnp.int32),
        pltpu.VMEM((K, PG, C), jnp.float32),
        pltpu.VMEM((PG, C), jnp.float32),
        pltpu.SemaphoreType.DMA,
    ],
)
def _gather_sum_kernel(srow, z, out, sidx, gbuf, acc, sem):
    core = lax.axis_index("c")
    sub = lax.axis_index("s")
    wid = core * 16 + sub

    def _block(j, _):
        jg = wid * NSUB_Q + j
        pltpu.sync_copy(srow.at[jg], sidx)
        for g in range(SUB // PG):
            handles = [
                pltpu.async_copy(z.at[sidx.at[k, pl.ds(g * PG, PG)]],
                                 gbuf.at[k], sem)
                for k in range(K)
            ]
            for h in handles:
                h.wait()

            def _row(r, _):
                for t in range(C // 16):
                    tsl = pl.ds(t * 16, 16)
                    v = gbuf[0, r, tsl]
                    for k in range(1, K):
                        v = v + gbuf[k, r, tsl]
                    acc[r, tsl] = v
                return ()
            lax.fori_loop(0, PG, _row, ())
            pltpu.sync_copy(
                acc, out.at[pl.ds(jg * SUB + g * PG, PG)])
        return ()
    lax.fori_loop(0, NSUB_Q, _block, ())


BN_D = 2048


def _stats_body(x_ref, o_ref):
    i = pl.program_id(0)

    @pl.when(i == 0)
    def _():
        o_ref[...] = jnp.zeros_like(o_ref)
    x = x_ref[...]
    o_ref[0:1, :] += jnp.sum(x, axis=0, keepdims=True)
    o_ref[1:2, :] += jnp.sum(x * x, axis=0, keepdims=True)


def _stats(out_pre):
    return pl.pallas_call(
        _stats_body,
        grid=(N_PAD_A // BN_D,),
        in_specs=[pl.BlockSpec((BN_D, C), lambda i: (i, 0))],
        out_specs=pl.BlockSpec((8, C), lambda i: (0, 0)),
        out_shape=jax.ShapeDtypeStruct((8, C), jnp.float32),
    )(out_pre)


def _bn_body(x_ref, s_ref, g_ref, b_ref, o_ref):
    x = x_ref[...]
    mean = s_ref[0:1, :] * (1.0 / N)
    var = s_ref[1:2, :] * (1.0 / N) - mean * mean
    rstd = lax.rsqrt(var + 1e-5)
    y = (x - mean) * (rstd * g_ref[...]) + b_ref[...]
    o_ref[...] = jnp.maximum(y, 0.0)


def _bn_relu(out_pre, stats, gamma, beta):
    return pl.pallas_call(
        _bn_body,
        grid=(N_PAD_A // BN_D,),
        in_specs=[
            pl.BlockSpec((BN_D, C), lambda i: (i, 0)),
            pl.BlockSpec((8, C), lambda i: (0, 0)),
            pl.BlockSpec((1, C), lambda i: (0, 0)),
            pl.BlockSpec((1, C), lambda i: (0, 0)),
        ],
        out_specs=pl.BlockSpec((BN_D, C), lambda i: (i, 0)),
        out_shape=jax.ShapeDtypeStruct((N_PAD_A, C), jnp.float32),
    )(out_pre, stats, gamma, beta)


def kernel(feats, coords, W, gamma, beta):
    ci = coords.astype(jnp.int32)
    cpad = jnp.pad(ci, ((0, N_PAD_A - N), (0, 0)))
    ct = cpad.T
    cb, cx, cy, cz = ct[0], ct[1], ct[2], ct[3]

    srow = _neighbors_kernel(cb, cx, cy, cz)

    feats_pad = jnp.pad(feats, ((0, N_PAD_B - N), (0, 0)))
    wcat = jnp.transpose(W, (1, 0, 2)).reshape(C, K * C)
    zmat = _gemm(feats_pad, wcat)
    z = zmat.reshape(Z_ROWS, C)

    out_pre = _gather_sum_kernel(srow, z)

    stats = _stats(out_pre)
    y = _bn_relu(out_pre, stats, gamma.reshape(1, C), beta.reshape(1, C))
    return y[:N]


# speedup vs baseline: 38.4461x; 1.1470x over previous
import functools

import jax
import jax.numpy as jnp
from jax import lax
from jax.experimental import pallas as pl
from jax.experimental.pallas import tpu as pltpu
from jax.experimental.pallas import tpu_sc as plsc

N = 50000
C = 128
K = 27
DD = 66
T = DD * DD * DD * 2
T_PAD = 576_000
TRASH = T

NW = 32
SUB = 128
N_PAD_A = 53248
CHUNK_Q = N_PAD_A // NW
NSUB_Q = CHUNK_Q // SUB
CHUNK_S = N_PAD_A // 16
NSUB_S = CHUNK_S // SUB
MEMSET_SP = T_PAD // 16
MBUF = 4000

BN_B = 1536
N_PAD_B = 50688
Z_ROWS = K * N_PAD_B

_DELTAS = [dx * DD * DD + dy * DD + dz
           for dx in (-1, 0, 1) for dy in (-1, 0, 1) for dz in (-1, 0, 1)]

_mesh = plsc.VectorSubcoreMesh(core_axis_name="c", subcore_axis_name="s")


def _key16(cb, cx, cy, cz, sl):
    b = cb[sl]
    x = cx[sl]
    y = cy[sl]
    z = cz[sl]
    return ((b * DD + x + 1) * DD + y + 1) * DD + z + 1


@functools.partial(
    pl.kernel,
    out_type=jax.ShapeDtypeStruct((N_PAD_A // SUB, K, SUB), jnp.int32),
    mesh=_mesh,
    scratch_types=[
        pltpu.VMEM((CHUNK_S,), jnp.int32),
        pltpu.VMEM((CHUNK_S,), jnp.int32),
        pltpu.VMEM((CHUNK_S,), jnp.int32),
        pltpu.VMEM((CHUNK_S,), jnp.int32),
        pltpu.VMEM((NSUB_S, SUB), jnp.int32),
        pltpu.VMEM((NSUB_S, SUB), jnp.int32),
        pltpu.VMEM((K, SUB), jnp.int32),
        pltpu.VMEM((K, SUB), jnp.int32),
        pltpu.VMEM((MBUF,), jnp.int32),
        pltpu.VMEM_SHARED((T_PAD,), jnp.int32),
        pltpu.SemaphoreType.DMA,
    ],
)
def _neighbors_kernel(cb, cx, cy, cz, srow_out,
                      cbv, cxv, cyv, czv, kidx, vals, qbuf, tbuf, mbuf,
                      table, sem):
    core = lax.axis_index("c")
    sub = lax.axis_index("s")
    wid = core * 16 + sub
    iota = lax.iota(jnp.int32, 16)

    neg1 = jnp.full((16,), -1, jnp.int32)

    def _fill(i, _):
        mbuf[pl.ds(i * 16, 16)] = neg1
        return ()
    lax.fori_loop(0, MBUF // 16, _fill, ())
    mbase = sub * MEMSET_SP

    def _memset(m, _):
        pltpu.sync_copy(mbuf, table.at[pl.ds(mbase + m * MBUF, MBUF)])
        return ()
    lax.fori_loop(0, MEMSET_SP // MBUF, _memset, ())

    sbase = sub * CHUNK_S
    pltpu.sync_copy(cb.at[pl.ds(sbase, CHUNK_S)], cbv)
    pltpu.sync_copy(cx.at[pl.ds(sbase, CHUNK_S)], cxv)
    pltpu.sync_copy(cy.at[pl.ds(sbase, CHUNK_S)], cyv)
    pltpu.sync_copy(cz.at[pl.ds(sbase, CHUNK_S)], czv)

    def _build(j, _):
        for t in range(SUB // 16):
            sl = pl.ds(j * SUB + t * 16, 16)
            key = _key16(cbv, cxv, cyv, czv, sl)
            gi = sbase + j * SUB + t * 16 + iota
            tsl = pl.ds(t * 16, 16)
            kidx[j, tsl] = jnp.where(gi < N, key, TRASH + (gi & 511))
            vals[j, tsl] = gi
        pltpu.sync_copy(vals.at[j], table.at[kidx.at[j]])
        return ()
    lax.fori_loop(0, NSUB_S, _build, ())

    plsc.subcore_barrier()

    wbase = wid * CHUNK_Q
    pltpu.sync_copy(cb.at[pl.ds(wbase, CHUNK_Q)], cbv.at[pl.ds(0, CHUNK_Q)])
    pltpu.sync_copy(cx.at[pl.ds(wbase, CHUNK_Q)], cxv.at[pl.ds(0, CHUNK_Q)])
    pltpu.sync_copy(cy.at[pl.ds(wbase, CHUNK_Q)], cyv.at[pl.ds(0, CHUNK_Q)])
    pltpu.sync_copy(cz.at[pl.ds(wbase, CHUNK_Q)], czv.at[pl.ds(0, CHUNK_Q)])

    def _query(j, _):
        for t in range(SUB // 16):
            sl = pl.ds(j * SUB + t * 16, 16)
            key = _key16(cbv, cxv, cyv, czv, sl)
            tsl = pl.ds(t * 16, 16)
            for k in range(K):
                qbuf[k, tsl] = key + _DELTAS[k]
        handles = [
            pltpu.async_copy(table.at[qbuf.at[k]], tbuf.at[k], sem)
            for k in range(K)
        ]
        for h in handles:
            h.wait()
        for t in range(SUB // 16):
            tsl = pl.ds(t * 16, 16)
            gi = wbase + j * SUB + t * 16 + iota
            pad = gi >= N
            for k in range(K):
                v = tbuf[k, tsl]
                zrow = k * N_PAD_B + N + ((gi + k * 131) & 511)
                tbuf[k, tsl] = jnp.where(pad | (v < 0), zrow,
                                         k * N_PAD_B + v)
        pltpu.sync_copy(tbuf, srow_out.at[wid * NSUB_Q + j])
        return ()
    lax.fori_loop(0, NSUB_Q, _query, ())


def _gemm_body(x_ref, w_ref, o_ref):
    o_ref[0] = jnp.dot(x_ref[...], w_ref[0],
                       preferred_element_type=jnp.float32)


def _gemm(feats_pad, w):
    return pl.pallas_call(
        _gemm_body,
        grid=(N_PAD_B // BN_B, K),
        in_specs=[
            pl.BlockSpec((BN_B, C), lambda i, k: (i, 0)),
            pl.BlockSpec((1, C, C), lambda i, k: (k, 0, 0)),
        ],
        out_specs=pl.BlockSpec((1, BN_B, C), lambda i, k: (k, i, 0)),
        out_shape=jax.ShapeDtypeStruct((K, N_PAD_B, C), jnp.float32),
    )(feats_pad, w)


PG = 32


@functools.partial(
    pl.kernel,
    out_type=jax.ShapeDtypeStruct((N_PAD_A, C), jnp.float32),
    mesh=_mesh,
    scratch_types=[
        pltpu.VMEM((K, SUB), jnp.int32),
        pltpu.VMEM((K, PG, C), jnp.float32),
        pltpu.VMEM((PG, C), jnp.float32),
        pltpu.SemaphoreType.DMA,
    ],
)
def _gather_sum_kernel(srow, z, out, sidx, gbuf, acc, sem):
    core = lax.axis_index("c")
    sub = lax.axis_index("s")
    wid = core * 16 + sub

    def _block(j, _):
        jg = wid * NSUB_Q + j
        pltpu.sync_copy(srow.at[jg], sidx)
        for g in range(SUB // PG):
            handles = [
                pltpu.async_copy(z.at[sidx.at[k, pl.ds(g * PG, PG)]],
                                 gbuf.at[k], sem)
                for k in range(K)
            ]
            for h in handles:
                h.wait()

            def _row(r, _):
                for t in range(C // 16):
                    tsl = pl.ds(t * 16, 16)
                    v = gbuf[0, r, tsl]
                    for k in range(1, K):
                        v = v + gbuf[k, r, tsl]
                    acc[r, tsl] = v
                return ()
            lax.fori_loop(0, PG, _row, ())
            pltpu.sync_copy(
                acc, out.at[pl.ds(jg * SUB + g * PG, PG)])
        return ()
    lax.fori_loop(0, NSUB_Q, _block, ())


BN_D = 2048


def _stats_body(x_ref, o_ref):
    i = pl.program_id(0)

    @pl.when(i == 0)
    def _():
        o_ref[...] = jnp.zeros_like(o_ref)
    x = x_ref[...]
    o_ref[0:1, :] += jnp.sum(x, axis=0, keepdims=True)
    o_ref[1:2, :] += jnp.sum(x * x, axis=0, keepdims=True)


def _stats(out_pre):
    return pl.pallas_call(
        _stats_body,
        grid=(N_PAD_A // BN_D,),
        in_specs=[pl.BlockSpec((BN_D, C), lambda i: (i, 0))],
        out_specs=pl.BlockSpec((8, C), lambda i: (0, 0)),
        out_shape=jax.ShapeDtypeStruct((8, C), jnp.float32),
    )(out_pre)


def _bn_body(x_ref, s_ref, g_ref, b_ref, o_ref):
    x = x_ref[...]
    mean = s_ref[0:1, :] * (1.0 / N)
    var = s_ref[1:2, :] * (1.0 / N) - mean * mean
    rstd = lax.rsqrt(var + 1e-5)
    y = (x - mean) * (rstd * g_ref[...]) + b_ref[...]
    o_ref[...] = jnp.maximum(y, 0.0)


def _bn_relu(out_pre, stats, gamma, beta):
    return pl.pallas_call(
        _bn_body,
        grid=(N_PAD_A // BN_D,),
        in_specs=[
            pl.BlockSpec((BN_D, C), lambda i: (i, 0)),
            pl.BlockSpec((8, C), lambda i: (0, 0)),
            pl.BlockSpec((1, C), lambda i: (0, 0)),
            pl.BlockSpec((1, C), lambda i: (0, 0)),
        ],
        out_specs=pl.BlockSpec((BN_D, C), lambda i: (i, 0)),
        out_shape=jax.ShapeDtypeStruct((N_PAD_A, C), jnp.float32),
    )(out_pre, stats, gamma, beta)


def kernel(feats, coords, W, gamma, beta):
    ci = coords.astype(jnp.int32)
    cpad = jnp.pad(ci, ((0, N_PAD_A - N), (0, 0)))
    ct = cpad.T
    cb, cx, cy, cz = ct[0], ct[1], ct[2], ct[3]

    srow = _neighbors_kernel(cb, cx, cy, cz)

    feats_pad = jnp.pad(feats, ((0, N_PAD_B - N), (0, 0)))
    z3 = _gemm(feats_pad, W)
    z = z3.reshape(Z_ROWS, C)

    out_pre = _gather_sum_kernel(srow, z)

    stats = _stats(out_pre)
    y = _bn_relu(out_pre, stats, gamma.reshape(1, C), beta.reshape(1, C))
    return y[:N]


# speedup vs baseline: 39.0666x; 1.0161x over previous
import functools

import jax
import jax.numpy as jnp
from jax import lax
from jax.experimental import pallas as pl
from jax.experimental.pallas import tpu as pltpu
from jax.experimental.pallas import tpu_sc as plsc

N = 50000
C = 128
K = 27
DD = 66
T = DD * DD * DD * 2
T_PAD = 576_000
TRASH = T

NW = 32
SUB = 128
N_PAD_A = 53248
CHUNK_Q = N_PAD_A // NW
NSUB_Q = CHUNK_Q // SUB
CHUNK_S = N_PAD_A // 16
NSUB_S = CHUNK_S // SUB
MEMSET_SP = T_PAD // 16
MBUF = 4000

BN_B = 1536
N_PAD_B = 50688
Z_ROWS = K * N_PAD_B

_DELTAS = [dx * DD * DD + dy * DD + dz
           for dx in (-1, 0, 1) for dy in (-1, 0, 1) for dz in (-1, 0, 1)]

_mesh = plsc.VectorSubcoreMesh(core_axis_name="c", subcore_axis_name="s")


def _key16(cb, cx, cy, cz, sl):
    b = cb[sl]
    x = cx[sl]
    y = cy[sl]
    z = cz[sl]
    return ((b * DD + x + 1) * DD + y + 1) * DD + z + 1


@functools.partial(
    pl.kernel,
    out_type=jax.ShapeDtypeStruct((N_PAD_A // SUB, K, SUB), jnp.int32),
    mesh=_mesh,
    scratch_types=[
        pltpu.VMEM((CHUNK_S,), jnp.int32),
        pltpu.VMEM((CHUNK_S,), jnp.int32),
        pltpu.VMEM((CHUNK_S,), jnp.int32),
        pltpu.VMEM((CHUNK_S,), jnp.int32),
        pltpu.VMEM((NSUB_S, SUB), jnp.int32),
        pltpu.VMEM((NSUB_S, SUB), jnp.int32),
        pltpu.VMEM((K, SUB), jnp.int32),
        pltpu.VMEM((K, SUB), jnp.int32),
        pltpu.VMEM((MBUF,), jnp.int32),
        pltpu.VMEM_SHARED((T_PAD,), jnp.int32),
        pltpu.SemaphoreType.DMA,
    ],
)
def _neighbors_kernel(cb, cx, cy, cz, srow_out,
                      cbv, cxv, cyv, czv, kidx, vals, qbuf, tbuf, mbuf,
                      table, sem):
    core = lax.axis_index("c")
    sub = lax.axis_index("s")
    wid = core * 16 + sub
    iota = lax.iota(jnp.int32, 16)

    neg1 = jnp.full((16,), -1, jnp.int32)

    def _fill(i, _):
        mbuf[pl.ds(i * 16, 16)] = neg1
        return ()
    lax.fori_loop(0, MBUF // 16, _fill, ())
    mbase = sub * MEMSET_SP

    def _memset(m, _):
        pltpu.sync_copy(mbuf, table.at[pl.ds(mbase + m * MBUF, MBUF)])
        return ()
    lax.fori_loop(0, MEMSET_SP // MBUF, _memset, ())

    sbase = sub * CHUNK_S
    pltpu.sync_copy(cb.at[pl.ds(sbase, CHUNK_S)], cbv)
    pltpu.sync_copy(cx.at[pl.ds(sbase, CHUNK_S)], cxv)
    pltpu.sync_copy(cy.at[pl.ds(sbase, CHUNK_S)], cyv)
    pltpu.sync_copy(cz.at[pl.ds(sbase, CHUNK_S)], czv)

    def _build(j, _):
        for t in range(SUB // 16):
            sl = pl.ds(j * SUB + t * 16, 16)
            key = _key16(cbv, cxv, cyv, czv, sl)
            gi = sbase + j * SUB + t * 16 + iota
            tsl = pl.ds(t * 16, 16)
            kidx[j, tsl] = jnp.where(gi < N, key, TRASH + (gi & 511))
            vals[j, tsl] = gi
        pltpu.sync_copy(vals.at[j], table.at[kidx.at[j]])
        return ()
    lax.fori_loop(0, NSUB_S, _build, ())

    plsc.subcore_barrier()

    wbase = wid * CHUNK_Q
    pltpu.sync_copy(cb.at[pl.ds(wbase, CHUNK_Q)], cbv.at[pl.ds(0, CHUNK_Q)])
    pltpu.sync_copy(cx.at[pl.ds(wbase, CHUNK_Q)], cxv.at[pl.ds(0, CHUNK_Q)])
    pltpu.sync_copy(cy.at[pl.ds(wbase, CHUNK_Q)], cyv.at[pl.ds(0, CHUNK_Q)])
    pltpu.sync_copy(cz.at[pl.ds(wbase, CHUNK_Q)], czv.at[pl.ds(0, CHUNK_Q)])

    def _query(j, _):
        for t in range(SUB // 16):
            sl = pl.ds(j * SUB + t * 16, 16)
            key = _key16(cbv, cxv, cyv, czv, sl)
            tsl = pl.ds(t * 16, 16)
            for k in range(K):
                qbuf[k, tsl] = key + _DELTAS[k]
        handles = [
            pltpu.async_copy(table.at[qbuf.at[k]], tbuf.at[k], sem)
            for k in range(K)
        ]
        for h in handles:
            h.wait()
        for t in range(SUB // 16):
            tsl = pl.ds(t * 16, 16)
            gi = wbase + j * SUB + t * 16 + iota
            pad = gi >= N
            for k in range(K):
                v = tbuf[k, tsl]
                zrow = k * N_PAD_B + N + ((gi + k * 131) & 511)
                tbuf[k, tsl] = jnp.where(pad | (v < 0), zrow,
                                         k * N_PAD_B + v)
        pltpu.sync_copy(tbuf, srow_out.at[wid * NSUB_Q + j])
        return ()
    lax.fori_loop(0, NSUB_Q, _query, ())


def _gemm_body(x_ref, w_ref, o_ref):
    o_ref[0] = jnp.dot(x_ref[...], w_ref[0],
                       preferred_element_type=jnp.float32)


def _gemm(feats_pad, w):
    return pl.pallas_call(
        _gemm_body,
        grid=(N_PAD_B // BN_B, K),
        in_specs=[
            pl.BlockSpec((BN_B, C), lambda i, k: (i, 0)),
            pl.BlockSpec((1, C, C), lambda i, k: (k, 0, 0)),
        ],
        out_specs=pl.BlockSpec((1, BN_B, C), lambda i, k: (k, i, 0)),
        out_shape=jax.ShapeDtypeStruct((K, N_PAD_B, C), jnp.float32),
    )(feats_pad, w)


PG = 32


@functools.partial(
    pl.kernel,
    out_type=jax.ShapeDtypeStruct((N_PAD_A, C), jnp.float32),
    mesh=_mesh,
    scratch_types=[
        pltpu.VMEM((K, SUB), jnp.int32),
        pltpu.VMEM((K, PG, C), jnp.float32),
        pltpu.VMEM((PG, C), jnp.float32),
        pltpu.SemaphoreType.DMA,
    ],
)
def _gather_sum_kernel(srow, z, out, sidx, gbuf, acc, sem):
    core = lax.axis_index("c")
    sub = lax.axis_index("s")
    wid = core * 16 + sub

    def _block(j, _):
        jg = wid * NSUB_Q + j
        pltpu.sync_copy(srow.at[jg], sidx)
        for g in range(SUB // PG):
            handles = [
                pltpu.async_copy(z.at[sidx.at[k, pl.ds(g * PG, PG)]],
                                 gbuf.at[k], sem)
                for k in range(K)
            ]
            for h in handles:
                h.wait()

            def _row(r, _):
                for t in range(C // 16):
                    tsl = pl.ds(t * 16, 16)
                    v = gbuf[0, r, tsl]
                    for k in range(1, K):
                        v = v + gbuf[k, r, tsl]
                    acc[r, tsl] = v
                return ()
            lax.fori_loop(0, PG, _row, ())
            pltpu.sync_copy(
                acc, out.at[pl.ds(jg * SUB + g * PG, PG)])
        return ()
    lax.fori_loop(0, NSUB_Q, _block, ())


BN_D = 2000


def _stats_body(x_ref, o_ref):
    i = pl.program_id(0)

    @pl.when(i == 0)
    def _():
        o_ref[...] = jnp.zeros_like(o_ref)
    x = x_ref[...]
    o_ref[0:1, :] += jnp.sum(x, axis=0, keepdims=True)
    o_ref[1:2, :] += jnp.sum(x * x, axis=0, keepdims=True)


def _stats(out_pre):
    return pl.pallas_call(
        _stats_body,
        grid=(N // BN_D,),
        in_specs=[pl.BlockSpec((BN_D, C), lambda i: (i, 0))],
        out_specs=pl.BlockSpec((8, C), lambda i: (0, 0)),
        out_shape=jax.ShapeDtypeStruct((8, C), jnp.float32),
    )(out_pre)


def _bn_body(x_ref, s_ref, g_ref, b_ref, o_ref):
    x = x_ref[...]
    mean = s_ref[0:1, :] * (1.0 / N)
    var = s_ref[1:2, :] * (1.0 / N) - mean * mean
    rstd = lax.rsqrt(var + 1e-5)
    y = (x - mean) * (rstd * g_ref[...]) + b_ref[...]
    o_ref[...] = jnp.maximum(y, 0.0)


def _bn_relu(out_pre, stats, gamma, beta):
    return pl.pallas_call(
        _bn_body,
        grid=(N // BN_D,),
        in_specs=[
            pl.BlockSpec((BN_D, C), lambda i: (i, 0)),
            pl.BlockSpec((8, C), lambda i: (0, 0)),
            pl.BlockSpec((1, C), lambda i: (0, 0)),
            pl.BlockSpec((1, C), lambda i: (0, 0)),
        ],
        out_specs=pl.BlockSpec((BN_D, C), lambda i: (i, 0)),
        out_shape=jax.ShapeDtypeStruct((N, C), jnp.float32),
    )(out_pre, stats, gamma, beta)


def kernel(feats, coords, W, gamma, beta):
    ci = coords.astype(jnp.int32)
    cpad = jnp.pad(ci, ((0, N_PAD_A - N), (0, 0)))
    ct = cpad.T
    cb, cx, cy, cz = ct[0], ct[1], ct[2], ct[3]

    srow = _neighbors_kernel(cb, cx, cy, cz)

    feats_pad = jnp.pad(feats, ((0, N_PAD_B - N), (0, 0)))
    z3 = _gemm(feats_pad, W)
    z = z3.reshape(Z_ROWS, C)

    out_pre = _gather_sum_kernel(srow, z)

    stats = _stats(out_pre)
    return _bn_relu(out_pre, stats, gamma.reshape(1, C), beta.reshape(1, C))


# speedup vs baseline: 43.9800x; 1.1258x over previous
import functools

import jax
import jax.numpy as jnp
from jax import lax
from jax.experimental import pallas as pl
from jax.experimental.pallas import tpu as pltpu
from jax.experimental.pallas import tpu_sc as plsc

N = 50000
C = 128
K = 27
DD = 66
T = DD * DD * DD * 2
T_PAD = 576_000
TRASH = T

NW = 32
SUB = 128
N_PAD_A = 53248
CHUNK_Q = N_PAD_A // NW
NSUB_Q = CHUNK_Q // SUB
CHUNK_S = N_PAD_A // 16
NSUB_S = CHUNK_S // SUB
MEMSET_SP = T_PAD // 16
MBUF = 4000

BN_B = 1536
N_PAD_B = 50688
Z_ROWS = K * N_PAD_B

_DELTAS = [dx * DD * DD + dy * DD + dz
           for dx in (-1, 0, 1) for dy in (-1, 0, 1) for dz in (-1, 0, 1)]

_mesh = plsc.VectorSubcoreMesh(core_axis_name="c", subcore_axis_name="s")


def _key16(cb, cx, cy, cz, sl):
    b = cb[sl]
    x = cx[sl]
    y = cy[sl]
    z = cz[sl]
    return ((b * DD + x + 1) * DD + y + 1) * DD + z + 1


@functools.partial(
    pl.kernel,
    out_type=jax.ShapeDtypeStruct((N_PAD_A // SUB, K, SUB), jnp.int32),
    mesh=_mesh,
    scratch_types=[
        pltpu.VMEM((CHUNK_S,), jnp.int32),
        pltpu.VMEM((CHUNK_S,), jnp.int32),
        pltpu.VMEM((CHUNK_S,), jnp.int32),
        pltpu.VMEM((CHUNK_S,), jnp.int32),
        pltpu.VMEM((NSUB_S, SUB), jnp.int32),
        pltpu.VMEM((NSUB_S, SUB), jnp.int32),
        pltpu.VMEM((K, SUB), jnp.int32),
        pltpu.VMEM((K, SUB), jnp.int32),
        pltpu.VMEM((MBUF,), jnp.int32),
        pltpu.VMEM_SHARED((T_PAD,), jnp.int32),
        pltpu.SemaphoreType.DMA,
    ],
)
def _neighbors_kernel(cb, cx, cy, cz, srow_out,
                      cbv, cxv, cyv, czv, kidx, vals, qbuf, tbuf, mbuf,
                      table, sem):
    core = lax.axis_index("c")
    sub = lax.axis_index("s")
    wid = core * 16 + sub
    iota = lax.iota(jnp.int32, 16)

    neg1 = jnp.full((16,), -1, jnp.int32)

    def _fill(i, _):
        mbuf[pl.ds(i * 16, 16)] = neg1
        return ()
    lax.fori_loop(0, MBUF // 16, _fill, ())
    mbase = sub * MEMSET_SP

    def _memset(m, _):
        pltpu.sync_copy(mbuf, table.at[pl.ds(mbase + m * MBUF, MBUF)])
        return ()
    lax.fori_loop(0, MEMSET_SP // MBUF, _memset, ())

    sbase = sub * CHUNK_S
    pltpu.sync_copy(cb.at[pl.ds(sbase, CHUNK_S)], cbv)
    pltpu.sync_copy(cx.at[pl.ds(sbase, CHUNK_S)], cxv)
    pltpu.sync_copy(cy.at[pl.ds(sbase, CHUNK_S)], cyv)
    pltpu.sync_copy(cz.at[pl.ds(sbase, CHUNK_S)], czv)

    def _build(j, _):
        for t in range(SUB // 16):
            sl = pl.ds(j * SUB + t * 16, 16)
            key = _key16(cbv, cxv, cyv, czv, sl)
            gi = sbase + j * SUB + t * 16 + iota
            tsl = pl.ds(t * 16, 16)
            kidx[j, tsl] = jnp.where(gi < N, key, TRASH + (gi & 511))
            vals[j, tsl] = gi
        pltpu.sync_copy(vals.at[j], table.at[kidx.at[j]])
        return ()
    lax.fori_loop(0, NSUB_S, _build, ())

    plsc.subcore_barrier()

    wbase = wid * CHUNK_Q
    pltpu.sync_copy(cb.at[pl.ds(wbase, CHUNK_Q)], cbv.at[pl.ds(0, CHUNK_Q)])
    pltpu.sync_copy(cx.at[pl.ds(wbase, CHUNK_Q)], cxv.at[pl.ds(0, CHUNK_Q)])
    pltpu.sync_copy(cy.at[pl.ds(wbase, CHUNK_Q)], cyv.at[pl.ds(0, CHUNK_Q)])
    pltpu.sync_copy(cz.at[pl.ds(wbase, CHUNK_Q)], czv.at[pl.ds(0, CHUNK_Q)])

    def _query(j, _):
        for t in range(SUB // 16):
            sl = pl.ds(j * SUB + t * 16, 16)
            key = _key16(cbv, cxv, cyv, czv, sl)
            tsl = pl.ds(t * 16, 16)
            for k in range(K):
                qbuf[k, tsl] = key + _DELTAS[k]
        handles = [
            pltpu.async_copy(table.at[qbuf.at[k]], tbuf.at[k], sem)
            for k in range(K)
        ]
        for h in handles:
            h.wait()
        for t in range(SUB // 16):
            tsl = pl.ds(t * 16, 16)
            gi = wbase + j * SUB + t * 16 + iota
            pad = gi >= N
            for k in range(K):
                v = tbuf[k, tsl]
                zrow = k * N_PAD_B + N + ((gi + k * 131) & 511)
                tbuf[k, tsl] = jnp.where(pad | (v < 0), zrow,
                                         k * N_PAD_B + v)
        pltpu.sync_copy(tbuf, srow_out.at[wid * NSUB_Q + j])
        return ()
    lax.fori_loop(0, NSUB_Q, _query, ())


def _gemm_body(x_ref, w_ref, o_ref):
    o_ref[0] = jnp.dot(x_ref[...], w_ref[0],
                       preferred_element_type=jnp.float32)


def _gemm(feats_pad, w):
    return pl.pallas_call(
        _gemm_body,
        grid=(N_PAD_B // BN_B, K),
        in_specs=[
            pl.BlockSpec((BN_B, C), lambda i, k: (i, 0)),
            pl.BlockSpec((1, C, C), lambda i, k: (k, 0, 0)),
        ],
        out_specs=pl.BlockSpec((1, BN_B, C), lambda i, k: (k, i, 0)),
        out_shape=jax.ShapeDtypeStruct((K, N_PAD_B, C), jnp.float32),
    )(feats_pad, w)


PG = 16
NG = SUB // PG


@functools.partial(
    pl.kernel,
    out_type=jax.ShapeDtypeStruct((N_PAD_A, C), jnp.float32),
    mesh=_mesh,
    scratch_types=[
        pltpu.VMEM((K, SUB), jnp.int32),
        pltpu.VMEM((2, K, PG, C), jnp.float32),
        pltpu.VMEM((PG, C), jnp.float32),
        pltpu.SemaphoreType.DMA,
    ],
)
def _gather_sum_kernel(srow, z, out, sidx, gbuf, acc, sem):
    core = lax.axis_index("c")
    sub = lax.axis_index("s")
    wid = core * 16 + sub

    def _fire(g, slot):
        return [
            pltpu.async_copy(z.at[sidx.at[k, pl.ds(g * PG, PG)]],
                             gbuf.at[slot, k], sem)
            for k in range(K)
        ]

    def _block(j, _):
        jg = wid * NSUB_Q + j
        pltpu.sync_copy(srow.at[jg], sidx)
        handles = _fire(0, 0)
        for g in range(NG):
            if g + 1 < NG:
                nxt = _fire(g + 1, (g + 1) % 2)
            for h in handles:
                h.wait()
            slot = g % 2

            def _row(r, _):
                for t in range(C // 16):
                    tsl = pl.ds(t * 16, 16)
                    v = gbuf[slot, 0, r, tsl]
                    for k in range(1, K):
                        v = v + gbuf[slot, k, r, tsl]
                    acc[r, tsl] = v
                return ()
            lax.fori_loop(0, PG, _row, ())
            pltpu.sync_copy(
                acc, out.at[pl.ds(jg * SUB + g * PG, PG)])
            if g + 1 < NG:
                handles = nxt
        return ()
    lax.fori_loop(0, NSUB_Q, _block, ())


BN_D = 2000


def _stats_body(x_ref, o_ref):
    i = pl.program_id(0)

    @pl.when(i == 0)
    def _():
        o_ref[...] = jnp.zeros_like(o_ref)
    x = x_ref[...]
    o_ref[0:1, :] += jnp.sum(x, axis=0, keepdims=True)
    o_ref[1:2, :] += jnp.sum(x * x, axis=0, keepdims=True)


def _stats(out_pre):
    return pl.pallas_call(
        _stats_body,
        grid=(N // BN_D,),
        in_specs=[pl.BlockSpec((BN_D, C), lambda i: (i, 0))],
        out_specs=pl.BlockSpec((8, C), lambda i: (0, 0)),
        out_shape=jax.ShapeDtypeStruct((8, C), jnp.float32),
    )(out_pre)


def _bn_body(x_ref, s_ref, g_ref, b_ref, o_ref):
    x = x_ref[...]
    mean = s_ref[0:1, :] * (1.0 / N)
    var = s_ref[1:2, :] * (1.0 / N) - mean * mean
    rstd = lax.rsqrt(var + 1e-5)
    y = (x - mean) * (rstd * g_ref[...]) + b_ref[...]
    o_ref[...] = jnp.maximum(y, 0.0)


def _bn_relu(out_pre, stats, gamma, beta):
    return pl.pallas_call(
        _bn_body,
        grid=(N // BN_D,),
        in_specs=[
            pl.BlockSpec((BN_D, C), lambda i: (i, 0)),
            pl.BlockSpec((8, C), lambda i: (0, 0)),
            pl.BlockSpec((1, C), lambda i: (0, 0)),
            pl.BlockSpec((1, C), lambda i: (0, 0)),
        ],
        out_specs=pl.BlockSpec((BN_D, C), lambda i: (i, 0)),
        out_shape=jax.ShapeDtypeStruct((N, C), jnp.float32),
    )(out_pre, stats, gamma, beta)


def kernel(feats, coords, W, gamma, beta):
    ci = coords.astype(jnp.int32)
    cpad = jnp.pad(ci, ((0, N_PAD_A - N), (0, 0)))
    ct = cpad.T
    cb, cx, cy, cz = ct[0], ct[1], ct[2], ct[3]

    srow = _neighbors_kernel(cb, cx, cy, cz)

    feats_pad = jnp.pad(feats, ((0, N_PAD_B - N), (0, 0)))
    z3 = _gemm(feats_pad, W)
    z = z3.reshape(Z_ROWS, C)

    out_pre = _gather_sum_kernel(srow, z)

    stats = _stats(out_pre)
    return _bn_relu(out_pre, stats, gamma.reshape(1, C), beta.reshape(1, C))


# speedup vs baseline: 43.9815x; 1.0000x over previous
import functools

import jax
import jax.numpy as jnp
from jax import lax
from jax.experimental import pallas as pl
from jax.experimental.pallas import tpu as pltpu
from jax.experimental.pallas import tpu_sc as plsc

N = 50000
C = 128
K = 27
DD = 66
T = DD * DD * DD * 2
T_PAD = 576_000
TRASH = T

NW = 32
SUB = 128
N_PAD_A = 53248
CHUNK_Q = N_PAD_A // NW
NSUB_Q = CHUNK_Q // SUB
CHUNK_S = N_PAD_A // 16
NSUB_S = CHUNK_S // SUB
MEMSET_SP = T_PAD // 16
MBUF = 4000

BN_B = 1536
N_PAD_B = 50688
Z_ROWS = K * N_PAD_B

_DELTAS = [dx * DD * DD + dy * DD + dz
           for dx in (-1, 0, 1) for dy in (-1, 0, 1) for dz in (-1, 0, 1)]

_mesh = plsc.VectorSubcoreMesh(core_axis_name="c", subcore_axis_name="s")


def _key16(cb, cx, cy, cz, sl):
    b = cb[sl]
    x = cx[sl]
    y = cy[sl]
    z = cz[sl]
    return ((b * DD + x + 1) * DD + y + 1) * DD + z + 1


@functools.partial(
    pl.kernel,
    out_type=jax.ShapeDtypeStruct((N_PAD_A // SUB, K, SUB), jnp.int32),
    mesh=_mesh,
    scratch_types=[
        pltpu.VMEM((CHUNK_S,), jnp.int32),
        pltpu.VMEM((CHUNK_S,), jnp.int32),
        pltpu.VMEM((CHUNK_S,), jnp.int32),
        pltpu.VMEM((CHUNK_S,), jnp.int32),
        pltpu.VMEM((NSUB_S, SUB), jnp.int32),
        pltpu.VMEM((NSUB_S, SUB), jnp.int32),
        pltpu.VMEM((K, SUB), jnp.int32),
        pltpu.VMEM((K, SUB), jnp.int32),
        pltpu.VMEM((MBUF,), jnp.int32),
        pltpu.VMEM_SHARED((T_PAD,), jnp.int32),
        pltpu.SemaphoreType.DMA,
    ],
)
def _neighbors_kernel(cb, cx, cy, cz, srow_out,
                      cbv, cxv, cyv, czv, kidx, vals, qbuf, tbuf, mbuf,
                      table, sem):
    core = lax.axis_index("c")
    sub = lax.axis_index("s")
    wid = core * 16 + sub
    iota = lax.iota(jnp.int32, 16)

    neg1 = jnp.full((16,), -1, jnp.int32)

    def _fill(i, _):
        mbuf[pl.ds(i * 16, 16)] = neg1
        return ()
    lax.fori_loop(0, MBUF // 16, _fill, ())
    mbase = sub * MEMSET_SP

    def _memset(m, _):
        pltpu.sync_copy(mbuf, table.at[pl.ds(mbase + m * MBUF, MBUF)])
        return ()
    lax.fori_loop(0, MEMSET_SP // MBUF, _memset, ())

    sbase = sub * CHUNK_S
    pltpu.sync_copy(cb.at[pl.ds(sbase, CHUNK_S)], cbv)
    pltpu.sync_copy(cx.at[pl.ds(sbase, CHUNK_S)], cxv)
    pltpu.sync_copy(cy.at[pl.ds(sbase, CHUNK_S)], cyv)
    pltpu.sync_copy(cz.at[pl.ds(sbase, CHUNK_S)], czv)

    def _build(j, _):
        for t in range(SUB // 16):
            sl = pl.ds(j * SUB + t * 16, 16)
            key = _key16(cbv, cxv, cyv, czv, sl)
            gi = sbase + j * SUB + t * 16 + iota
            tsl = pl.ds(t * 16, 16)
            kidx[j, tsl] = jnp.where(gi < N, key, TRASH + (gi & 511))
            vals[j, tsl] = gi
        pltpu.sync_copy(vals.at[j], table.at[kidx.at[j]])
        return ()
    lax.fori_loop(0, NSUB_S, _build, ())

    plsc.subcore_barrier()

    wbase = wid * CHUNK_Q
    pltpu.sync_copy(cb.at[pl.ds(wbase, CHUNK_Q)], cbv.at[pl.ds(0, CHUNK_Q)])
    pltpu.sync_copy(cx.at[pl.ds(wbase, CHUNK_Q)], cxv.at[pl.ds(0, CHUNK_Q)])
    pltpu.sync_copy(cy.at[pl.ds(wbase, CHUNK_Q)], cyv.at[pl.ds(0, CHUNK_Q)])
    pltpu.sync_copy(cz.at[pl.ds(wbase, CHUNK_Q)], czv.at[pl.ds(0, CHUNK_Q)])

    def _query(j, _):
        for t in range(SUB // 16):
            sl = pl.ds(j * SUB + t * 16, 16)
            key = _key16(cbv, cxv, cyv, czv, sl)
            tsl = pl.ds(t * 16, 16)
            for k in range(K):
                qbuf[k, tsl] = key + _DELTAS[k]
        handles = [
            pltpu.async_copy(table.at[qbuf.at[k]], tbuf.at[k], sem)
            for k in range(K)
        ]
        for h in handles:
            h.wait()
        for t in range(SUB // 16):
            tsl = pl.ds(t * 16, 16)
            gi = wbase + j * SUB + t * 16 + iota
            pad = gi >= N
            for k in range(K):
                v = tbuf[k, tsl]
                zrow = k * N_PAD_B + N + ((gi + k * 131) & 511)
                tbuf[k, tsl] = jnp.where(pad | (v < 0), zrow,
                                         k * N_PAD_B + v)
        pltpu.sync_copy(tbuf, srow_out.at[wid * NSUB_Q + j])
        return ()
    lax.fori_loop(0, NSUB_Q, _query, ())


def _gemm_body(x_ref, w_ref, o_ref):
    o_ref[0] = jnp.dot(x_ref[...].astype(jnp.bfloat16),
                       w_ref[0].astype(jnp.bfloat16),
                       preferred_element_type=jnp.float32)


def _gemm(feats_pad, w):
    return pl.pallas_call(
        _gemm_body,
        grid=(N_PAD_B // BN_B, K),
        in_specs=[
            pl.BlockSpec((BN_B, C), lambda i, k: (i, 0)),
            pl.BlockSpec((1, C, C), lambda i, k: (k, 0, 0)),
        ],
        out_specs=pl.BlockSpec((1, BN_B, C), lambda i, k: (k, i, 0)),
        out_shape=jax.ShapeDtypeStruct((K, N_PAD_B, C), jnp.float32),
    )(feats_pad, w)


PG = 16
NG = SUB // PG


@functools.partial(
    pl.kernel,
    out_type=jax.ShapeDtypeStruct((N_PAD_A, C), jnp.float32),
    mesh=_mesh,
    scratch_types=[
        pltpu.VMEM((K, SUB), jnp.int32),
        pltpu.VMEM((2, K, PG, C), jnp.float32),
        pltpu.VMEM((PG, C), jnp.float32),
        pltpu.SemaphoreType.DMA,
    ],
)
def _gather_sum_kernel(srow, z, out, sidx, gbuf, acc, sem):
    core = lax.axis_index("c")
    sub = lax.axis_index("s")
    wid = core * 16 + sub

    def _fire(g, slot):
        return [
            pltpu.async_copy(z.at[sidx.at[k, pl.ds(g * PG, PG)]],
                             gbuf.at[slot, k], sem)
            for k in range(K)
        ]

    def _block(j, _):
        jg = wid * NSUB_Q + j
        pltpu.sync_copy(srow.at[jg], sidx)
        handles = _fire(0, 0)
        for g in range(NG):
            if g + 1 < NG:
                nxt = _fire(g + 1, (g + 1) % 2)
            for h in handles:
                h.wait()
            slot = g % 2

            def _row(r, _):
                for t in range(C // 16):
                    tsl = pl.ds(t * 16, 16)
                    v = gbuf[slot, 0, r, tsl]
                    for k in range(1, K):
                        v = v + gbuf[slot, k, r, tsl]
                    acc[r, tsl] = v
                return ()
            lax.fori_loop(0, PG, _row, ())
            pltpu.sync_copy(
                acc, out.at[pl.ds(jg * SUB + g * PG, PG)])
            if g + 1 < NG:
                handles = nxt
        return ()
    lax.fori_loop(0, NSUB_Q, _block, ())


BN_D = 2000


def _stats_body(x_ref, o_ref):
    i = pl.program_id(0)

    @pl.when(i == 0)
    def _():
        o_ref[...] = jnp.zeros_like(o_ref)
    x = x_ref[...]
    o_ref[0:1, :] += jnp.sum(x, axis=0, keepdims=True)
    o_ref[1:2, :] += jnp.sum(x * x, axis=0, keepdims=True)


def _stats(out_pre):
    return pl.pallas_call(
        _stats_body,
        grid=(N // BN_D,),
        in_specs=[pl.BlockSpec((BN_D, C), lambda i: (i, 0))],
        out_specs=pl.BlockSpec((8, C), lambda i: (0, 0)),
        out_shape=jax.ShapeDtypeStruct((8, C), jnp.float32),
    )(out_pre)


def _bn_body(x_ref, s_ref, g_ref, b_ref, o_ref):
    x = x_ref[...]
    mean = s_ref[0:1, :] * (1.0 / N)
    var = s_ref[1:2, :] * (1.0 / N) - mean * mean
    rstd = lax.rsqrt(var + 1e-5)
    y = (x - mean) * (rstd * g_ref[...]) + b_ref[...]
    o_ref[...] = jnp.maximum(y, 0.0)


def _bn_relu(out_pre, stats, gamma, beta):
    return pl.pallas_call(
        _bn_body,
        grid=(N // BN_D,),
        in_specs=[
            pl.BlockSpec((BN_D, C), lambda i: (i, 0)),
            pl.BlockSpec((8, C), lambda i: (0, 0)),
            pl.BlockSpec((1, C), lambda i: (0, 0)),
            pl.BlockSpec((1, C), lambda i: (0, 0)),
        ],
        out_specs=pl.BlockSpec((BN_D, C), lambda i: (i, 0)),
        out_shape=jax.ShapeDtypeStruct((N, C), jnp.float32),
    )(out_pre, stats, gamma, beta)


def kernel(feats, coords, W, gamma, beta):
    ci = coords.astype(jnp.int32)
    cpad = jnp.pad(ci, ((0, N_PAD_A - N), (0, 0)))
    ct = cpad.T
    cb, cx, cy, cz = ct[0], ct[1], ct[2], ct[3]

    srow = _neighbors_kernel(cb, cx, cy, cz)

    feats_pad = jnp.pad(feats, ((0, N_PAD_B - N), (0, 0)))
    z3 = _gemm(feats_pad, W)
    z = z3.reshape(Z_ROWS, C)

    out_pre = _gather_sum_kernel(srow, z)

    stats = _stats(out_pre)
    return _bn_relu(out_pre, stats, gamma.reshape(1, C), beta.reshape(1, C))
